# Initial kernel scaffold; baseline (speedup 1.0000x reference)
import jax
import jax.numpy as jnp
from jax.experimental import pallas as pl


def kernel(x, adj, W_pre, W_att):
    raise NotImplementedError("write your pallas kernel here")


# speedup vs baseline: 1.5289x; 1.5289x over previous
import functools

import jax
import jax.numpy as jnp
from jax import lax
from jax.experimental import pallas as pl
from jax.experimental.pallas import tpu as pltpu

_ALPHA = 0.2


def _prep_kernel(x_ref, wpre_ref, watt_ref,
                 h_ref, es_ref, Es_ref, Es2_ref, er_ref, Er_ref, Er2_ref):
    x = x_ref[...]
    h = jnp.dot(x, wpre_ref[...], preferred_element_type=jnp.float32)
    h_ref[...] = h
    u = h.shape[1]
    a = watt_ref[...]
    e_s = jnp.dot(h, a[:u, :], preferred_element_type=jnp.float32)
    e_r = jnp.dot(h, a[u:, :], preferred_element_type=jnp.float32)
    m = jnp.max(e_s)
    es_ref[...] = e_s
    Es_ref[...] = jnp.exp(e_s - m)
    Es2_ref[...] = jnp.exp(_ALPHA * (e_s - m))
    er_ref[...] = e_r
    t = m + e_r
    c = jnp.where(t > 0.0, t, _ALPHA * t)
    Er_ref[...] = jnp.exp(e_r + m - c)
    Er2_ref[...] = jnp.exp(_ALPHA * (e_r + m) - c)


def _gat_kernel(nblk, adj_ref, h_ref, es_ref, Es_ref, Es2_ref, rvec_ref,
                out_ref):
    j = pl.program_id(0)
    n, rb = adj_ref.shape
    a = adj_ref[...]
    rows = lax.broadcasted_iota(jnp.int32, (n, rb), 0)
    cols = lax.broadcasted_iota(jnp.int32, (n, rb), 1) + j * rb
    rv = rvec_ref[...]
    er_b = rv[0:1, :]
    Er_b = rv[1:2, :]
    Er2_b = rv[2:3, :]
    mask = jnp.logical_or(a > 0.0, rows == cols)
    tpos = (es_ref[...] + er_b) > 0.0
    p = jnp.where(tpos, Es_ref[...] * Er_b, Es2_ref[...] * Er2_b)
    p = jnp.where(mask, p, 0.0)
    colsum = jnp.sum(p, axis=0, keepdims=True)
    pn = p * (1.0 / colsum)
    hb = h_ref[pl.ds(j * rb, rb), :]
    contrib = jnp.dot(pn, hb, preferred_element_type=jnp.float32)

    @pl.when(j == 0)
    def _():
        out_ref[...] = contrib

    @pl.when(j != 0)
    def _():
        out_ref[...] = out_ref[...] + contrib

    @pl.when(j == nblk - 1)
    def _():
        o = out_ref[...]
        out_ref[...] = jnp.where(o > 0.0, o, jnp.exp(o) - 1.0)


def _build_calls(n, d, units, interpret=False):
    prep = pl.pallas_call(
        _prep_kernel,
        out_shape=[jax.ShapeDtypeStruct((n, units), jnp.float32)]
        + [jax.ShapeDtypeStruct((n, 1), jnp.float32)] * 6,
        interpret=interpret,
    )
    rb = 512 if n % 512 == 0 else n
    nblk = n // rb
    main = pl.pallas_call(
        functools.partial(_gat_kernel, nblk),
        grid=(nblk,),
        in_specs=[
            pl.BlockSpec((n, rb), lambda j: (0, j)),
            pl.BlockSpec((n, units), lambda j: (0, 0)),
            pl.BlockSpec((n, 1), lambda j: (0, 0)),
            pl.BlockSpec((n, 1), lambda j: (0, 0)),
            pl.BlockSpec((n, 1), lambda j: (0, 0)),
            pl.BlockSpec((8, rb), lambda j: (0, j)),
        ],
        out_specs=pl.BlockSpec((n, units), lambda j: (0, 0)),
        out_shape=jax.ShapeDtypeStruct((n, units), jnp.float32),
        compiler_params=pltpu.CompilerParams(
            dimension_semantics=("arbitrary",)),
        interpret=interpret,
    )
    return prep, main


def kernel(x, adj, W_pre, W_att):
    b, n, d = x.shape
    units = W_pre.shape[1]
    prep, main = _build_calls(n, d, units)
    h, es, Es, Es2, er, Er, Er2 = prep(x[0], W_pre, W_att)
    rvec = jnp.concatenate(
        [er.T, Er.T, Er2.T, jnp.zeros((5, n), jnp.float32)], axis=0)
    out = main(adj[0], h, es, Es, Es2, rvec)
    return out[None]


# speedup vs baseline: 1.9200x; 1.2558x over previous
import functools

import jax
import jax.numpy as jnp
from jax import lax
from jax.experimental import pallas as pl
from jax.experimental.pallas import tpu as pltpu

_ALPHA = 0.2


def _prep_kernel(x_ref, wpre_ref, watt_ref,
                 h_ref, Es_ref, Es2_ref, Er_ref, Er2_ref):
    x = x_ref[...]
    h = jnp.dot(x, wpre_ref[...], preferred_element_type=jnp.float32)
    h_ref[...] = h
    u = h.shape[1]
    a = watt_ref[...]
    e_s = jnp.dot(h, a[:u, :], preferred_element_type=jnp.float32)
    e_r = jnp.dot(h, a[u:, :], preferred_element_type=jnp.float32)
    m = jnp.max(e_s)
    Es_ref[...] = jnp.exp(e_s - m)
    Es2_ref[...] = jnp.exp(_ALPHA * (e_s - m))
    t = m + e_r
    c = jnp.where(t > 0.0, t, _ALPHA * t)
    Er_ref[...] = jnp.exp(e_r + m - c)
    Er2_ref[...] = jnp.exp(_ALPHA * (e_r + m) - c)


def _gat_kernel(nblk, adj_ref, h_ref, Es_ref, Es2_ref, rvec_ref, out_ref):
    j = pl.program_id(0)
    n, rb = adj_ref.shape
    a = adj_ref[...]
    rv = rvec_ref[...]
    Er_b = rv[0:1, :]
    Er2_b = rv[1:2, :]
    p = a * jnp.maximum(Es_ref[...] * Er_b, Es2_ref[...] * Er2_b)
    colsum = jnp.sum(p, axis=0, keepdims=True)
    sub = adj_ref[pl.ds(j * rb, rb), :]
    eye = (lax.broadcasted_iota(jnp.int32, (rb, rb), 0)
           == lax.broadcasted_iota(jnp.int32, (rb, rb), 1)).astype(jnp.float32)
    Es_b = Es_ref[pl.ds(j * rb, rb), :]
    Es2_b = Es2_ref[pl.ds(j * rb, rb), :]
    msel_sub = jnp.maximum(Es_b * Er_b, Es2_b * Er2_b)
    dmat = eye * (1.0 - sub) * msel_sub
    colsum = colsum + jnp.sum(dmat, axis=0, keepdims=True)
    inv = (1.0 / colsum).reshape(rb, 1)
    hb = h_ref[pl.ds(j * rb, rb), :]
    hs = hb * inv
    contrib = jnp.dot(p, hs, preferred_element_type=jnp.float32)

    @pl.when(j == 0)
    def _():
        out_ref[...] = contrib

    @pl.when(j != 0)
    def _():
        out_ref[...] = out_ref[...] + contrib

    dvals = jnp.sum(dmat, axis=1, keepdims=True)
    out_ref[pl.ds(j * rb, rb), :] = out_ref[pl.ds(j * rb, rb), :] + dvals * hs

    @pl.when(j == nblk - 1)
    def _():
        o = out_ref[...]
        out_ref[...] = jnp.where(o > 0.0, o, jnp.exp(o) - 1.0)


def _build_calls(n, d, units, interpret=False):
    prep = pl.pallas_call(
        _prep_kernel,
        out_shape=[jax.ShapeDtypeStruct((n, units), jnp.float32)]
        + [jax.ShapeDtypeStruct((n, 1), jnp.float32)] * 4,
        interpret=interpret,
    )
    rb = 512 if n % 512 == 0 else n
    nblk = n // rb
    main = pl.pallas_call(
        functools.partial(_gat_kernel, nblk),
        grid=(nblk,),
        in_specs=[
            pl.BlockSpec((n, rb), lambda j: (0, j)),
            pl.BlockSpec((n, units), lambda j: (0, 0)),
            pl.BlockSpec((n, 1), lambda j: (0, 0)),
            pl.BlockSpec((n, 1), lambda j: (0, 0)),
            pl.BlockSpec((8, rb), lambda j: (0, j)),
        ],
        out_specs=pl.BlockSpec((n, units), lambda j: (0, 0)),
        out_shape=jax.ShapeDtypeStruct((n, units), jnp.float32),
        compiler_params=pltpu.CompilerParams(
            dimension_semantics=("arbitrary",)),
        interpret=interpret,
    )
    return prep, main


def kernel(x, adj, W_pre, W_att):
    b, n, d = x.shape
    units = W_pre.shape[1]
    prep, main = _build_calls(n, d, units)
    h, Es, Es2, Er, Er2 = prep(x[0], W_pre, W_att)
    rvec = jnp.concatenate(
        [Er.T, Er2.T, jnp.zeros((6, n), jnp.float32)], axis=0)
    out = main(adj[0], h, Es, Es2, rvec)
    return out[None]


# speedup vs baseline: 2.2608x; 1.1775x over previous
import functools

import jax
import jax.numpy as jnp
from jax import lax
from jax.experimental import pallas as pl
from jax.experimental.pallas import tpu as pltpu

_ALPHA = 0.2


def _prep_kernel(x_ref, wpre_ref, watt_ref,
                 h_ref, Esb_ref, Es2b_ref, rvec_ref):
    x = x_ref[...]
    h = jnp.dot(x, wpre_ref[...], preferred_element_type=jnp.float32)
    h_ref[...] = h
    u = h.shape[1]
    a = watt_ref[...]
    e_s = jnp.dot(h, a[:u, :], preferred_element_type=jnp.float32)
    m = jnp.max(e_s)
    ones_row = jnp.ones((1, u), jnp.float32)
    Esb_ref[...] = jnp.exp(e_s - m) * ones_row
    Es2b_ref[...] = jnp.exp(_ALPHA * (e_s - m)) * ones_row
    e_r = lax.dot_general(a[u:, :], h, (((0,), (1,)), ((), ())),
                          preferred_element_type=jnp.float32)
    t = m + e_r
    c = jnp.where(t > 0.0, t, _ALPHA * t)
    n = h.shape[0]
    rvec_ref[0:1, :] = jnp.exp(e_r + m - c)
    rvec_ref[1:2, :] = jnp.exp(_ALPHA * (e_r + m) - c)
    rvec_ref[2:8, :] = jnp.zeros((6, n), jnp.float32)


def _gat_kernel(nblk, adj_ref, h_ref, Esb_ref, Es2b_ref, rvec_ref, out_ref,
                p_buf):
    j = pl.program_id(0)
    n, rb = adj_ref.shape
    Esb = Esb_ref[...]
    Es2b = Es2b_ref[...]
    eye = (lax.broadcasted_iota(jnp.int32, (128, 128), 0)
           == lax.broadcasted_iota(jnp.int32, (128, 128), 1)
           ).astype(jnp.float32)
    cs_parts, dval_parts = [], []
    for g in range(rb // 128):
        lo = g * 128
        a_g = adj_ref[:, lo:lo + 128]
        Er_g = rvec_ref[0:1, lo:lo + 128]
        Er2_g = rvec_ref[1:2, lo:lo + 128]
        p_g = a_g * jnp.maximum(Esb * Er_g, Es2b * Er2_g)
        p_buf[:, lo:lo + 128] = p_g
        cs_g = jnp.sum(p_g, axis=0, keepdims=True)
        row0 = j * rb + lo
        sub = adj_ref[pl.ds(row0, 128), lo:lo + 128]
        msel = jnp.maximum(Esb_ref[pl.ds(row0, 128), :] * Er_g,
                           Es2b_ref[pl.ds(row0, 128), :] * Er2_g)
        dmat = eye * (1.0 - sub) * msel
        cs_parts.append(cs_g + jnp.sum(dmat, axis=0, keepdims=True))
        dval_parts.append(jnp.sum(dmat, axis=1, keepdims=True))
    colsum = jnp.concatenate(cs_parts, axis=1)
    inv = (1.0 / colsum).reshape(rb, 1)
    hb = h_ref[pl.ds(j * rb, rb), :]
    hs = hb * inv
    contrib = jnp.dot(p_buf[...], hs, preferred_element_type=jnp.float32)

    @pl.when(j == 0)
    def _():
        out_ref[...] = contrib

    @pl.when(j != 0)
    def _():
        out_ref[...] = out_ref[...] + contrib

    dvals = jnp.concatenate(dval_parts, axis=0)
    out_ref[pl.ds(j * rb, rb), :] = (
        out_ref[pl.ds(j * rb, rb), :] + dvals * hs)

    @pl.when(j == nblk - 1)
    def _():
        o = out_ref[...]
        out_ref[...] = jnp.where(o > 0.0, o, jnp.exp(o) - 1.0)


def _build_calls(n, d, units, interpret=False):
    prep = pl.pallas_call(
        _prep_kernel,
        out_shape=[
            jax.ShapeDtypeStruct((n, units), jnp.float32),
            jax.ShapeDtypeStruct((n, units), jnp.float32),
            jax.ShapeDtypeStruct((n, units), jnp.float32),
            jax.ShapeDtypeStruct((8, n), jnp.float32),
        ],
        interpret=interpret,
    )
    rb = 512 if n % 512 == 0 else n
    nblk = n // rb
    main = pl.pallas_call(
        functools.partial(_gat_kernel, nblk),
        grid=(nblk,),
        in_specs=[
            pl.BlockSpec((n, rb), lambda j: (0, j)),
            pl.BlockSpec((n, units), lambda j: (0, 0)),
            pl.BlockSpec((n, units), lambda j: (0, 0)),
            pl.BlockSpec((n, units), lambda j: (0, 0)),
            pl.BlockSpec((8, rb), lambda j: (0, j)),
        ],
        out_specs=pl.BlockSpec((n, units), lambda j: (0, 0)),
        out_shape=jax.ShapeDtypeStruct((n, units), jnp.float32),
        scratch_shapes=[pltpu.VMEM((n, rb), jnp.float32)],
        compiler_params=pltpu.CompilerParams(
            dimension_semantics=("arbitrary",)),
        interpret=interpret,
    )
    return prep, main


def kernel(x, adj, W_pre, W_att):
    b, n, d = x.shape
    units = W_pre.shape[1]
    prep, main = _build_calls(n, d, units)
    h, Esb, Es2b, rvec = prep(x[0], W_pre, W_att)
    out = main(adj[0], h, Esb, Es2b, rvec)
    return out[None]
